# Initial kernel scaffold; baseline (speedup 1.0000x reference)
import jax
import jax.numpy as jnp
from jax.experimental import pallas as pl


def kernel(feat_0, feat_1, feat_2, feat_3, feat_4, feat_5, feat_6, feat_7, feat_8, feat_9, feat_10, feat_11, feat_12, feat_13, feat_14, feat_15, feat_16, feat_17, feat_18, feat_19, feat_20, feat_21, feat_22, feat_23, feat_24, feat_25, W_0, W_1, W_2, W_3, W_4, W_5, W_6, W_7, W_8, W_9, W_10, W_11, W_12, W_13, W_14, W_15, W_16, W_17, W_18, W_19, W_20, W_21, W_22, W_23, W_24, W_25):
    raise NotImplementedError("write your pallas kernel here")


# speedup vs baseline: 3.6812x; 3.6812x over previous
import functools

import jax
import jax.numpy as jnp
from jax import lax
from jax.experimental import pallas as pl
from jax.experimental.pallas import tpu as pltpu
from jax.experimental.pallas import tpu_sc as plsc

DIM = 128
BATCH = 4096
N_FIELDS = 26


def _build():
    info = plsc.get_sparse_core_info()
    nc, ns = info.num_cores, info.num_subcores
    nw = nc * ns
    bpw = BATCH // nw

    mesh = plsc.VectorSubcoreMesh(core_axis_name="c", subcore_axis_name="s")
    out_type = tuple(
        jax.ShapeDtypeStruct((BATCH, DIM), jnp.float32) for _ in range(N_FIELDS)
    )

    @functools.partial(
        pl.kernel,
        mesh=mesh,
        out_type=out_type,
        scratch_types=[
            pltpu.VMEM((N_FIELDS, bpw), jnp.int32),
            pltpu.VMEM((bpw, DIM), jnp.float32),
            pltpu.SemaphoreType.DMA,
        ],
    )
    def emb_kernel(*refs):
        feats = refs[:N_FIELDS]
        tables = refs[N_FIELDS : 2 * N_FIELDS]
        outs = refs[2 * N_FIELDS : 3 * N_FIELDS]
        idx_v, rows_v, sem = refs[3 * N_FIELDS :]

        wid = lax.axis_index("s") * nc + lax.axis_index("c")
        base = wid * bpw

        for i in range(N_FIELDS):
            pltpu.sync_copy(feats[i].at[pl.ds(base, bpw)], idx_v.at[i])
        for i in range(N_FIELDS):
            pltpu.async_copy(tables[i].at[idx_v.at[i]], rows_v, sem).wait()
            pltpu.sync_copy(rows_v, outs[i].at[pl.ds(base, bpw)])

    return emb_kernel


_emb_kernel = _build()


def kernel(
    feat_0, feat_1, feat_2, feat_3, feat_4, feat_5, feat_6, feat_7,
    feat_8, feat_9, feat_10, feat_11, feat_12, feat_13, feat_14, feat_15,
    feat_16, feat_17, feat_18, feat_19, feat_20, feat_21, feat_22, feat_23,
    feat_24, feat_25,
    W_0, W_1, W_2, W_3, W_4, W_5, W_6, W_7,
    W_8, W_9, W_10, W_11, W_12, W_13, W_14, W_15,
    W_16, W_17, W_18, W_19, W_20, W_21, W_22, W_23,
    W_24, W_25,
):
    args = locals()
    feats = [args[f"feat_{i}"] for i in range(N_FIELDS)]
    tables = [args[f"W_{i}"] for i in range(N_FIELDS)]
    outs = _emb_kernel(*feats, *tables)
    return tuple(o.reshape(BATCH, 1, DIM) for o in outs)


# speedup vs baseline: 5.4081x; 1.4691x over previous
import functools

import jax
import jax.numpy as jnp
from jax import lax
from jax.experimental import pallas as pl
from jax.experimental.pallas import tpu as pltpu
from jax.experimental.pallas import tpu_sc as plsc

DIM = 128
BATCH = 4096
N_FIELDS = 26


def _build():
    info = plsc.get_sparse_core_info()
    nc, ns = info.num_cores, info.num_subcores
    nw = nc * ns
    bpw = BATCH // nw

    depth = 4

    mesh = plsc.VectorSubcoreMesh(core_axis_name="c", subcore_axis_name="s")
    out_type = tuple(
        jax.ShapeDtypeStruct((BATCH, DIM), jnp.float32) for _ in range(N_FIELDS)
    )

    @functools.partial(
        pl.kernel,
        mesh=mesh,
        out_type=out_type,
        scratch_types=[
            pltpu.VMEM((N_FIELDS, bpw), jnp.int32),
            *[pltpu.VMEM((bpw, DIM), jnp.float32) for _ in range(depth)],
            pltpu.SemaphoreType.DMA,
            *[pltpu.SemaphoreType.DMA for _ in range(depth)],
            *[pltpu.SemaphoreType.DMA for _ in range(depth)],
        ],
    )
    def emb_kernel(*refs):
        feats = refs[:N_FIELDS]
        tables = refs[N_FIELDS : 2 * N_FIELDS]
        outs = refs[2 * N_FIELDS : 3 * N_FIELDS]
        scratch = refs[3 * N_FIELDS :]
        idx_v = scratch[0]
        rows = scratch[1 : 1 + depth]
        isem = scratch[1 + depth]
        gsems = scratch[2 + depth : 2 + 2 * depth]
        ssems = scratch[2 + 2 * depth :]

        wid = lax.axis_index("s") * nc + lax.axis_index("c")
        base = wid * bpw

        icps = [
            pltpu.async_copy(feats[i].at[pl.ds(base, bpw)], idx_v.at[i], isem)
            for i in range(N_FIELDS)
        ]
        for cp in icps:
            cp.wait()

        def fire_gather(i):
            b = i % depth
            return pltpu.async_copy(tables[i].at[idx_v.at[i]], rows[b], gsems[b])

        def fire_store(i):
            b = i % depth
            return pltpu.async_copy(
                rows[b], outs[i].at[pl.ds(base, bpw)], ssems[b]
            )

        gcps = [None] * N_FIELDS
        scps = [None] * N_FIELDS
        for j in range(min(depth - 1, N_FIELDS)):
            gcps[j] = fire_gather(j)
        for i in range(N_FIELDS):
            if i >= 1:
                scps[i - 1].wait()
            j = i + depth - 1
            if j < N_FIELDS:
                gcps[j] = fire_gather(j)
            gcps[i].wait()
            scps[i] = fire_store(i)
        scps[N_FIELDS - 1].wait()

    return emb_kernel


_emb_kernel = _build()


def kernel(
    feat_0, feat_1, feat_2, feat_3, feat_4, feat_5, feat_6, feat_7,
    feat_8, feat_9, feat_10, feat_11, feat_12, feat_13, feat_14, feat_15,
    feat_16, feat_17, feat_18, feat_19, feat_20, feat_21, feat_22, feat_23,
    feat_24, feat_25,
    W_0, W_1, W_2, W_3, W_4, W_5, W_6, W_7,
    W_8, W_9, W_10, W_11, W_12, W_13, W_14, W_15,
    W_16, W_17, W_18, W_19, W_20, W_21, W_22, W_23,
    W_24, W_25,
):
    args = locals()
    feats = [args[f"feat_{i}"] for i in range(N_FIELDS)]
    tables = [args[f"W_{i}"] for i in range(N_FIELDS)]
    outs = _emb_kernel(*feats, *tables)
    return tuple(o.reshape(BATCH, 1, DIM) for o in outs)


# speedup vs baseline: 5.5337x; 1.0232x over previous
import functools

import jax
import jax.numpy as jnp
from jax import lax
from jax.experimental import pallas as pl
from jax.experimental.pallas import tpu as pltpu
from jax.experimental.pallas import tpu_sc as plsc

DIM = 128
BATCH = 4096
N_FIELDS = 26


def _build():
    info = plsc.get_sparse_core_info()
    nc, ns = info.num_cores, info.num_subcores
    nw = nc * ns
    bpw = BATCH // nw

    depth = 6

    mesh = plsc.VectorSubcoreMesh(core_axis_name="c", subcore_axis_name="s")
    out_type = tuple(
        jax.ShapeDtypeStruct((BATCH, DIM), jnp.float32) for _ in range(N_FIELDS)
    )

    @functools.partial(
        pl.kernel,
        mesh=mesh,
        out_type=out_type,
        scratch_types=[
            pltpu.VMEM((N_FIELDS, bpw), jnp.int32),
            *[pltpu.VMEM((bpw, DIM), jnp.float32) for _ in range(depth)],
            pltpu.SemaphoreType.DMA,
            *[pltpu.SemaphoreType.DMA for _ in range(depth)],
            *[pltpu.SemaphoreType.DMA for _ in range(depth)],
        ],
    )
    def emb_kernel(*refs):
        feats = refs[:N_FIELDS]
        tables = refs[N_FIELDS : 2 * N_FIELDS]
        outs = refs[2 * N_FIELDS : 3 * N_FIELDS]
        scratch = refs[3 * N_FIELDS :]
        idx_v = scratch[0]
        rows = scratch[1 : 1 + depth]
        isem = scratch[1 + depth]
        gsems = scratch[2 + depth : 2 + 2 * depth]
        ssems = scratch[2 + 2 * depth :]

        wid = lax.axis_index("s") * nc + lax.axis_index("c")
        base = wid * bpw

        icps = [
            pltpu.async_copy(feats[i].at[pl.ds(base, bpw)], idx_v.at[i], isem)
            for i in range(N_FIELDS)
        ]
        for cp in icps:
            cp.wait()

        def fire_gather(i):
            b = i % depth
            return pltpu.async_copy(tables[i].at[idx_v.at[i]], rows[b], gsems[b])

        def fire_store(i):
            b = i % depth
            return pltpu.async_copy(
                rows[b], outs[i].at[pl.ds(base, bpw)], ssems[b]
            )

        gcps = [None] * N_FIELDS
        scps = [None] * N_FIELDS
        for j in range(min(depth - 1, N_FIELDS)):
            gcps[j] = fire_gather(j)
        for i in range(N_FIELDS):
            if i >= 1:
                scps[i - 1].wait()
            j = i + depth - 1
            if j < N_FIELDS:
                gcps[j] = fire_gather(j)
            gcps[i].wait()
            scps[i] = fire_store(i)
        scps[N_FIELDS - 1].wait()

    return emb_kernel


_emb_kernel = _build()


def kernel(
    feat_0, feat_1, feat_2, feat_3, feat_4, feat_5, feat_6, feat_7,
    feat_8, feat_9, feat_10, feat_11, feat_12, feat_13, feat_14, feat_15,
    feat_16, feat_17, feat_18, feat_19, feat_20, feat_21, feat_22, feat_23,
    feat_24, feat_25,
    W_0, W_1, W_2, W_3, W_4, W_5, W_6, W_7,
    W_8, W_9, W_10, W_11, W_12, W_13, W_14, W_15,
    W_16, W_17, W_18, W_19, W_20, W_21, W_22, W_23,
    W_24, W_25,
):
    args = locals()
    feats = [args[f"feat_{i}"] for i in range(N_FIELDS)]
    tables = [args[f"W_{i}"] for i in range(N_FIELDS)]
    outs = _emb_kernel(*feats, *tables)
    return tuple(o.reshape(BATCH, 1, DIM) for o in outs)


# speedup vs baseline: 5.8539x; 1.0579x over previous
import functools

import jax
import jax.numpy as jnp
from jax import lax
from jax.experimental import pallas as pl
from jax.experimental.pallas import tpu as pltpu
from jax.experimental.pallas import tpu_sc as plsc

DIM = 128
BATCH = 4096
N_FIELDS = 26


def _build():
    info = plsc.get_sparse_core_info()
    nc, ns = info.num_cores, info.num_subcores
    nw = nc * ns
    bpw = BATCH // nw

    depth = 3
    n_staged = 9
    small_vocab = 1000
    staged_fields = list(range(6, 6 + n_staged))
    hbm_fields = [f for f in range(N_FIELDS) if f not in staged_fields]
    order = hbm_fields + staged_fields
    first_staged_pos = len(hbm_fields)

    mesh = plsc.VectorSubcoreMesh(core_axis_name="c", subcore_axis_name="s")
    out_type = tuple(
        jax.ShapeDtypeStruct((BATCH, DIM), jnp.float32) for _ in range(N_FIELDS)
    )

    @functools.partial(
        pl.kernel,
        mesh=mesh,
        out_type=out_type,
        scratch_types=[
            pltpu.VMEM((N_FIELDS, bpw), jnp.int32),
            *[pltpu.VMEM((bpw, DIM), jnp.float32) for _ in range(depth)],
            *[
                pltpu.VMEM_SHARED((small_vocab, DIM), jnp.float32)
                for _ in range(n_staged)
            ],
            pltpu.SemaphoreType.DMA,
            *[pltpu.SemaphoreType.DMA for _ in range(depth)],
            *[pltpu.SemaphoreType.DMA for _ in range(depth)],
        ],
    )
    def emb_kernel(*refs):
        feats = refs[:N_FIELDS]
        tables = refs[N_FIELDS : 2 * N_FIELDS]
        outs = refs[2 * N_FIELDS : 3 * N_FIELDS]
        scratch = refs[3 * N_FIELDS :]
        idx_v = scratch[0]
        rows = scratch[1 : 1 + depth]
        shared = scratch[1 + depth : 1 + depth + n_staged]
        isem = scratch[1 + depth + n_staged]
        gsems = scratch[2 + depth + n_staged : 2 + 2 * depth + n_staged]
        ssems = scratch[2 + 2 * depth + n_staged :]

        sid = lax.axis_index("s")
        wid = sid * nc + lax.axis_index("c")
        base = wid * bpw

        icps = [
            pltpu.async_copy(feats[i].at[pl.ds(base, bpw)], idx_v.at[i], isem)
            for i in range(N_FIELDS)
        ]

        for t in range(n_staged):

            @pl.when(sid == t)
            def _(t=t):
                pltpu.sync_copy(tables[staged_fields[t]], shared[t])

        for cp in icps:
            cp.wait()

        def fire_gather(pos):
            f = order[pos]
            b = pos % depth
            src = (
                shared[f - 6].at[idx_v.at[f]]
                if f in staged_fields
                else tables[f].at[idx_v.at[f]]
            )
            return pltpu.async_copy(src, rows[b], gsems[b])

        def fire_store(pos):
            f = order[pos]
            b = pos % depth
            return pltpu.async_copy(
                rows[b], outs[f].at[pl.ds(base, bpw)], ssems[b]
            )

        gcps = [None] * N_FIELDS
        scps = [None] * N_FIELDS
        for j in range(min(depth - 1, N_FIELDS)):
            gcps[j] = fire_gather(j)
        for i in range(N_FIELDS):
            if i >= 1:
                scps[i - 1].wait()
            j = i + depth - 1
            if j < N_FIELDS:
                if j == first_staged_pos:
                    plsc.subcore_barrier()
                gcps[j] = fire_gather(j)
            gcps[i].wait()
            scps[i] = fire_store(i)
        scps[N_FIELDS - 1].wait()

    return emb_kernel


_emb_kernel = _build()


def kernel(
    feat_0, feat_1, feat_2, feat_3, feat_4, feat_5, feat_6, feat_7,
    feat_8, feat_9, feat_10, feat_11, feat_12, feat_13, feat_14, feat_15,
    feat_16, feat_17, feat_18, feat_19, feat_20, feat_21, feat_22, feat_23,
    feat_24, feat_25,
    W_0, W_1, W_2, W_3, W_4, W_5, W_6, W_7,
    W_8, W_9, W_10, W_11, W_12, W_13, W_14, W_15,
    W_16, W_17, W_18, W_19, W_20, W_21, W_22, W_23,
    W_24, W_25,
):
    args = locals()
    feats = [args[f"feat_{i}"] for i in range(N_FIELDS)]
    tables = [args[f"W_{i}"] for i in range(N_FIELDS)]
    outs = _emb_kernel(*feats, *tables)
    return tuple(o.reshape(BATCH, 1, DIM) for o in outs)
